# baseline (speedup 1.0000x reference)
import functools

import jax
import jax.numpy as jnp
from jax.experimental import pallas as pl
from jax.experimental.pallas import tpu as pltpu

_K = 40


def _bn(x, g, b, eps=1e-5):
    axes = (0,) + tuple(range(2, x.ndim))
    m = jnp.mean(x, axis=axes, keepdims=True)
    v = jnp.var(x, axis=axes, keepdims=True)
    shape = (1, -1) + (1,) * (x.ndim - 2)
    return g.reshape(shape) * (x - m) / jnp.sqrt(v + eps) + b.reshape(shape)


def _conv(x, W):
    if x.ndim == 3:
        return jnp.einsum("oi,bin->bon", W, x)
    return jnp.einsum("oi,bihw->bohw", W, x)


def _block(x, p, name):
    y = _conv(x, p["W" + name])
    y = _bn(y, p["g" + name], p["b" + name])
    return jax.nn.leaky_relu(y, 0.2)


def _knn(x, k):
    inner = -2.0 * jnp.matmul(jnp.swapaxes(x, 2, 1), x)
    xx = jnp.sum(x ** 2, axis=1, keepdims=True)
    pd = -xx - inner - jnp.swapaxes(xx, 2, 1)
    return jax.lax.top_k(pd, k)[1]


def _graph_feature(x, k):
    B, C, N = x.shape
    idx = _knn(x, k)
    xt = jnp.swapaxes(x, 2, 1)
    idx_base = jnp.arange(B).reshape(-1, 1, 1) * N
    flat_idx = (idx + idx_base).reshape(-1)
    feature = xt.reshape(B * N, C)[flat_idx].reshape(B, N, k, C)
    xe = jnp.broadcast_to(xt[:, :, None, :], (B, N, k, C))
    f = jnp.concatenate((feature - xe, xe), axis=3)
    return jnp.transpose(f, (0, 3, 1, 2))


def _final_proj_kernel(h_ref, w_ref, o_ref):
    o_ref[0] = jnp.dot(w_ref[...], h_ref[0],
                       preferred_element_type=jnp.float32)


def _final_proj(h, W):
    B, C, N = h.shape
    O = W.shape[0]
    Opad = 56
    Wp = jnp.zeros((Opad, C), W.dtype).at[:O].set(W)
    out = pl.pallas_call(
        _final_proj_kernel,
        grid=(B,),
        in_specs=[
            pl.BlockSpec((1, C, N), lambda b: (b, 0, 0)),
            pl.BlockSpec((Opad, C), lambda b: (0, 0)),
        ],
        out_specs=pl.BlockSpec((1, Opad, N), lambda b: (b, 0, 0)),
        out_shape=jax.ShapeDtypeStruct((B, Opad, N), jnp.float32),
    )(h, Wp)
    return out[:, :O, :]


def _forward(x, p):
    B, _, N = x.shape
    x2z = _block(x[:, :2, :], p, "0_0") * x[:, 2, :][:, None, :]
    x2y = _block(x[:, jnp.array([0, 2]), :], p, "0_1") * x[:, 1, :][:, None, :]
    x2x = _block(x[:, 1:3, :], p, "0_1") * x[:, 0, :][:, None, :]
    h = jnp.concatenate((x, x2x, x2y, x2z), axis=1)
    f = _graph_feature(h, _K)
    f = _block(f, p, "1")
    f = _block(f, p, "2")
    x1 = jnp.max(f, axis=-1)
    f = _graph_feature(x1, _K)
    f = _block(f, p, "3")
    f = _block(f, p, "4")
    x2 = jnp.max(f, axis=-1)
    f = _graph_feature(x2, _K)
    f = _block(f, p, "5")
    x3 = jnp.max(f, axis=-1)
    c = jnp.concatenate((x1, x2, x3), axis=1)
    e = _block(c, p, "6")
    e = jnp.max(e, axis=-1, keepdims=True)
    e = jnp.repeat(e, N, axis=2)
    h = jnp.concatenate((e, x1, x2, x3), axis=1)
    h = _block(h, p, "7")
    h = _block(h, p, "8")
    h = _block(h, p, "9")
    return _final_proj(h, p["W10"])


def kernel(x, params):
    return _forward(x, params)


# speedup vs baseline: 1.2964x; 1.2964x over previous
import functools

import jax
import jax.numpy as jnp
from jax import lax
from jax.experimental import pallas as pl
from jax.experimental.pallas import tpu as pltpu
from jax.experimental.pallas import tpu_sc as plsc

_K = 40
_NC, _NS = 2, 16
_NW = _NC * _NS


def _sc_gather_call(table, idx, ch):
    M = idx.shape[0]
    R, Cp = table.shape
    m_per_w = M // _NW
    assert m_per_w % ch == 0 and m_per_w % 8 == 0
    n_steps = m_per_w // ch
    mesh = plsc.VectorSubcoreMesh(core_axis_name="c", subcore_axis_name="s",
                                  num_cores=_NC, num_subcores=_NS)

    @functools.partial(
        pl.kernel,
        out_type=jax.ShapeDtypeStruct((M, Cp), jnp.float32),
        mesh=mesh,
        scratch_types=[
            pltpu.VMEM((ch,), jnp.int32),
            pltpu.VMEM((ch, Cp), jnp.float32),
            pltpu.SemaphoreType.DMA,
        ],
        compiler_params=pltpu.CompilerParams(use_tc_tiling_on_sc=False),
    )
    def gather_k(table_hbm, idx_hbm, out_hbm, idx_v, rows_v, sem):
        wid = lax.axis_index("s") * _NC + lax.axis_index("c")
        w_base = wid * m_per_w

        def step(i, carry):
            base = w_base + i * ch
            pltpu.sync_copy(idx_hbm.at[pl.ds(base, ch)], idx_v)
            pltpu.async_copy(table_hbm.at[idx_v], rows_v, sem).wait()
            pltpu.sync_copy(rows_v, out_hbm.at[pl.ds(base, ch)])
            return carry

        lax.fori_loop(0, n_steps, step, 0)

    return gather_k(table, idx)


def _bn(x, g, b, eps=1e-5):
    axes = (0,) + tuple(range(2, x.ndim))
    m = jnp.mean(x, axis=axes, keepdims=True)
    v = jnp.var(x, axis=axes, keepdims=True)
    shape = (1, -1) + (1,) * (x.ndim - 2)
    return g.reshape(shape) * (x - m) / jnp.sqrt(v + eps) + b.reshape(shape)


def _conv(x, W):
    if x.ndim == 3:
        return jnp.einsum("oi,bin->bon", W, x)
    return jnp.einsum("oi,bihw->bohw", W, x)


def _block(x, p, name):
    y = _conv(x, p["W" + name])
    y = _bn(y, p["g" + name], p["b" + name])
    return jax.nn.leaky_relu(y, 0.2)


def _knn(x, k):
    inner = -2.0 * jnp.matmul(jnp.swapaxes(x, 2, 1), x)
    xx = jnp.sum(x ** 2, axis=1, keepdims=True)
    pd = -xx - inner - jnp.swapaxes(xx, 2, 1)
    return jax.lax.top_k(pd, k)[1]


def _graph_feature(x, k):
    B, C, N = x.shape
    idx = _knn(x, k)
    xt = jnp.swapaxes(x, 2, 1)
    idx_base = jnp.arange(B).reshape(-1, 1, 1) * N
    flat_idx = (idx + idx_base).reshape(-1).astype(jnp.int32)
    Cp = max(16, ((C + 15) // 16) * 16)
    table = jnp.zeros((B * N, Cp), jnp.float32).at[:, :C].set(
        xt.reshape(B * N, C))
    feature = _sc_gather_call(table, flat_idx, 1024)[:, :C]
    feature = feature.reshape(B, N, k, C)
    xe = jnp.broadcast_to(xt[:, :, None, :], (B, N, k, C))
    f = jnp.concatenate((feature - xe, xe), axis=3)
    return jnp.transpose(f, (0, 3, 1, 2))


def _final_proj_kernel(h_ref, w_ref, o_ref):
    o_ref[0] = jnp.dot(w_ref[...], h_ref[0],
                       preferred_element_type=jnp.float32)


def _final_proj(h, W):
    B, C, N = h.shape
    O = W.shape[0]
    Opad = 56
    Wp = jnp.zeros((Opad, C), W.dtype).at[:O].set(W)
    out = pl.pallas_call(
        _final_proj_kernel,
        grid=(B,),
        in_specs=[
            pl.BlockSpec((1, C, N), lambda b: (b, 0, 0)),
            pl.BlockSpec((Opad, C), lambda b: (0, 0)),
        ],
        out_specs=pl.BlockSpec((1, Opad, N), lambda b: (b, 0, 0)),
        out_shape=jax.ShapeDtypeStruct((B, Opad, N), jnp.float32),
    )(h, Wp)
    return out[:, :O, :]


def _forward(x, p):
    B, _, N = x.shape
    x2z = _block(x[:, :2, :], p, "0_0") * x[:, 2, :][:, None, :]
    x2y = _block(x[:, jnp.array([0, 2]), :], p, "0_1") * x[:, 1, :][:, None, :]
    x2x = _block(x[:, 1:3, :], p, "0_1") * x[:, 0, :][:, None, :]
    h = jnp.concatenate((x, x2x, x2y, x2z), axis=1)
    f = _graph_feature(h, _K)
    f = _block(f, p, "1")
    f = _block(f, p, "2")
    x1 = jnp.max(f, axis=-1)
    f = _graph_feature(x1, _K)
    f = _block(f, p, "3")
    f = _block(f, p, "4")
    x2 = jnp.max(f, axis=-1)
    f = _graph_feature(x2, _K)
    f = _block(f, p, "5")
    x3 = jnp.max(f, axis=-1)
    c = jnp.concatenate((x1, x2, x3), axis=1)
    e = _block(c, p, "6")
    e = jnp.max(e, axis=-1, keepdims=True)
    e = jnp.repeat(e, N, axis=2)
    h = jnp.concatenate((e, x1, x2, x3), axis=1)
    h = _block(h, p, "7")
    h = _block(h, p, "8")
    h = _block(h, p, "9")
    return _final_proj(h, p["W10"])


def kernel(x, params):
    return _forward(x, params)


# speedup vs baseline: 3.3755x; 2.6037x over previous
import functools

import jax
import jax.numpy as jnp
from jax import lax
from jax.experimental import pallas as pl
from jax.experimental.pallas import tpu as pltpu
from jax.experimental.pallas import tpu_sc as plsc

_K = 40
_NC, _NS = 2, 16
_NW = _NC * _NS
_BR = 256


def _sc_gather_call(table, idx, ch):
    M = idx.shape[0]
    R, Cp = table.shape
    m_per_w = M // _NW
    assert m_per_w % ch == 0 and m_per_w % 8 == 0
    n_steps = m_per_w // ch
    mesh = plsc.VectorSubcoreMesh(core_axis_name="c", subcore_axis_name="s",
                                  num_cores=_NC, num_subcores=_NS)

    @functools.partial(
        pl.kernel,
        out_type=jax.ShapeDtypeStruct((M, Cp), jnp.float32),
        mesh=mesh,
        scratch_types=[
            pltpu.VMEM((ch,), jnp.int32),
            pltpu.VMEM((ch, Cp), jnp.float32),
            pltpu.SemaphoreType.DMA,
        ],
        compiler_params=pltpu.CompilerParams(use_tc_tiling_on_sc=False),
    )
    def gather_k(table_hbm, idx_hbm, out_hbm, idx_v, rows_v, sem):
        wid = lax.axis_index("s") * _NC + lax.axis_index("c")
        w_base = wid * m_per_w

        def step(i, carry):
            base = w_base + i * ch
            pltpu.sync_copy(idx_hbm.at[pl.ds(base, ch)], idx_v)
            pltpu.async_copy(table_hbm.at[idx_v], rows_v, sem).wait()
            pltpu.sync_copy(rows_v, out_hbm.at[pl.ds(base, ch)])
            return carry

        lax.fori_loop(0, n_steps, step, 0)

    return gather_k(table, idx)


def _bisect_body(pd_ref, pmat_ref, o_ref):
    pd = pd_ref[...]
    kb = jax.lax.bitcast_convert_type(pd, jnp.int32)
    key = jnp.where(kb < 0, kb ^ jnp.int32(0x7FFFFFFF), kb)

    def it(_, lohi):
        lo, hi = lohi
        mid = (lo & hi) + ((lo ^ hi) >> 1)
        cnt = jnp.sum((key > mid).astype(jnp.int32), axis=1, keepdims=True)
        ge = cnt >= _K
        return jnp.where(ge, mid, lo), jnp.where(ge, hi, mid)

    lo0 = jnp.full((_BR, 1), jnp.iinfo(jnp.int32).min, jnp.int32)
    hi0 = jnp.full((_BR, 1), jnp.iinfo(jnp.int32).max, jnp.int32)
    _, hi = jax.lax.fori_loop(0, 32, it, (lo0, hi0))
    strict = (key > hi).astype(jnp.float32)
    eq = (key == hi).astype(jnp.float32)
    g = jnp.sum(strict, axis=1, keepdims=True).astype(jnp.int32)
    sw = jnp.dot(strict, pmat_ref[...],
                 preferred_element_type=jnp.float32).astype(jnp.int32)
    ew = jnp.dot(eq, pmat_ref[...],
                 preferred_element_type=jnp.float32).astype(jnp.int32)
    o_ref[...] = jnp.concatenate(
        [sw, ew, jnp.broadcast_to(g, (_BR, 128))], axis=1)


def _sc_emit_call(words):
    R = words.shape[0]
    rows_per_w = R // _NW
    RCH = 16
    n_ch = rows_per_w // RCH
    mesh = plsc.VectorSubcoreMesh(core_axis_name="c", subcore_axis_name="s",
                                  num_cores=_NC, num_subcores=_NS)

    @functools.partial(
        pl.kernel,
        out_type=jax.ShapeDtypeStruct((R, 48), jnp.int32),
        mesh=mesh,
        scratch_types=[
            pltpu.VMEM((RCH, 384), jnp.int32),
            pltpu.VMEM((RCH, 48), jnp.int32),
        ],
        compiler_params=pltpu.CompilerParams(use_tc_tiling_on_sc=False,
                                             needs_layout_passes=False),
    )
    def emit_k(words_hbm, out_hbm, win_v, out_v):
        wid = lax.axis_index("s") * _NC + lax.axis_index("c")
        w_base = wid * rows_per_w
        lane = lax.iota(jnp.int32, 16)

        def decode_pass(r, col_off, cnt0, limit):
            def grp(g, cnt):
                w0 = win_v[r, pl.ds(col_off + g * 16, 16)]
                colbase = (g * 16 + lane) * 16

                def popc(u):
                    v = u - ((u >> 1) & 0x5555)
                    v = (v & 0x3333) + ((v >> 2) & 0x3333)
                    v = (v + (v >> 4)) & 0x0F0F
                    return (v + (v >> 8)) & 0x1F

                def with_bits():
                    wcnt = popc(w0)
                    prefix = cnt + plsc.cumsum(wcnt) - wcnt
                    for b in range(16):
                        m = ((w0 >> b) & 1) != 0
                        acc = popc(w0 & ((1 << b) - 1))
                        pos = prefix + acc
                        plsc.store_scatter(
                            out_v,
                            [jnp.full((16,), r, jnp.int32), pos],
                            colbase + b,
                            mask=jnp.logical_and(m, pos < limit))
                    return cnt + jnp.sum(wcnt)

                nz = jnp.sum((w0 != 0).astype(jnp.int32))
                return lax.cond(nz > 0, with_bits, lambda: cnt)

            return lax.fori_loop(0, 8, grp, cnt0)

        def chunk(ci, carry):
            base = w_base + ci * RCH
            pltpu.sync_copy(words_hbm.at[pl.ds(base, RCH)], win_v)

            def row(r, carry2):
                g_r = win_v[r, pl.ds(256, 16)][0]
                decode_pass(r, 0, 0, jnp.int32(48))
                decode_pass(r, 128, g_r, jnp.int32(_K))
                return carry2

            lax.fori_loop(0, RCH, row, 0)
            pltpu.sync_copy(out_v, out_hbm.at[pl.ds(base, RCH)])
            return carry

        lax.fori_loop(0, n_ch, chunk, 0)

    return emit_k(words)


def _knn(x, k):
    B, C, N = x.shape
    inner = -2.0 * jnp.matmul(jnp.swapaxes(x, 2, 1), x)
    xx = jnp.sum(x ** 2, axis=1, keepdims=True)
    pd = (-xx - inner - jnp.swapaxes(xx, 2, 1)).reshape(B * N, N)
    NB = (B * N) // _BR
    col = jax.lax.broadcasted_iota(jnp.int32, (N, 128), 0)
    wrd = jax.lax.broadcasted_iota(jnp.int32, (N, 128), 1)
    pmat = jnp.where(col // 16 == wrd, 1 << (col % 16), 0).astype(jnp.float32)
    words = pl.pallas_call(
        _bisect_body,
        grid=(NB,),
        in_specs=[
            pl.BlockSpec((_BR, N), lambda r: (r, 0)),
            pl.BlockSpec((N, 128), lambda r: (0, 0)),
        ],
        out_specs=pl.BlockSpec((_BR, 384), lambda r: (r, 0)),
        out_shape=jax.ShapeDtypeStruct((B * N, 384), jnp.int32),
    )(pd, pmat)
    idx = _sc_emit_call(words)[:, :k]
    v = jnp.take_along_axis(pd, idx, axis=1)
    kb2 = jax.lax.bitcast_convert_type(v, jnp.int32)
    vkey = jnp.where(kb2 < 0, kb2 ^ jnp.int32(0x7FFFFFFF), kb2)
    _, idx = jax.lax.sort((-vkey, idx), dimension=1, num_keys=2,
                          is_stable=False)
    return idx.reshape(B, N, k)


def _bn(x, g, b, eps=1e-5):
    axes = (0,) + tuple(range(2, x.ndim))
    m = jnp.mean(x, axis=axes, keepdims=True)
    v = jnp.var(x, axis=axes, keepdims=True)
    shape = (1, -1) + (1,) * (x.ndim - 2)
    return g.reshape(shape) * (x - m) / jnp.sqrt(v + eps) + b.reshape(shape)


def _conv(x, W):
    if x.ndim == 3:
        return jnp.einsum("oi,bin->bon", W, x)
    return jnp.einsum("oi,bihw->bohw", W, x)


def _block(x, p, name):
    y = _conv(x, p["W" + name])
    y = _bn(y, p["g" + name], p["b" + name])
    return jax.nn.leaky_relu(y, 0.2)


def _graph_feature(x, k):
    B, C, N = x.shape
    idx = _knn(x, k)
    xt = jnp.swapaxes(x, 2, 1)
    idx_base = jnp.arange(B).reshape(-1, 1, 1) * N
    flat_idx = (idx + idx_base).reshape(-1).astype(jnp.int32)
    Cp = max(16, ((C + 15) // 16) * 16)
    table = jnp.zeros((B * N, Cp), jnp.float32).at[:, :C].set(
        xt.reshape(B * N, C))
    feature = _sc_gather_call(table, flat_idx, 1024)[:, :C]
    feature = feature.reshape(B, N, k, C)
    xe = jnp.broadcast_to(xt[:, :, None, :], (B, N, k, C))
    f = jnp.concatenate((feature - xe, xe), axis=3)
    return jnp.transpose(f, (0, 3, 1, 2))


def _final_proj_kernel(h_ref, w_ref, o_ref):
    o_ref[0] = jnp.dot(w_ref[...], h_ref[0],
                       preferred_element_type=jnp.float32)


def _final_proj(h, W):
    B, C, N = h.shape
    O = W.shape[0]
    Opad = 56
    Wp = jnp.zeros((Opad, C), W.dtype).at[:O].set(W)
    out = pl.pallas_call(
        _final_proj_kernel,
        grid=(B,),
        in_specs=[
            pl.BlockSpec((1, C, N), lambda b: (b, 0, 0)),
            pl.BlockSpec((Opad, C), lambda b: (0, 0)),
        ],
        out_specs=pl.BlockSpec((1, Opad, N), lambda b: (b, 0, 0)),
        out_shape=jax.ShapeDtypeStruct((B, Opad, N), jnp.float32),
    )(h, Wp)
    return out[:, :O, :]


def _forward(x, p):
    B, _, N = x.shape
    x2z = _block(x[:, :2, :], p, "0_0") * x[:, 2, :][:, None, :]
    x2y = _block(x[:, jnp.array([0, 2]), :], p, "0_1") * x[:, 1, :][:, None, :]
    x2x = _block(x[:, 1:3, :], p, "0_1") * x[:, 0, :][:, None, :]
    h = jnp.concatenate((x, x2x, x2y, x2z), axis=1)
    f = _graph_feature(h, _K)
    f = _block(f, p, "1")
    f = _block(f, p, "2")
    x1 = jnp.max(f, axis=-1)
    f = _graph_feature(x1, _K)
    f = _block(f, p, "3")
    f = _block(f, p, "4")
    x2 = jnp.max(f, axis=-1)
    f = _graph_feature(x2, _K)
    f = _block(f, p, "5")
    x3 = jnp.max(f, axis=-1)
    c = jnp.concatenate((x1, x2, x3), axis=1)
    e = _block(c, p, "6")
    e = jnp.max(e, axis=-1, keepdims=True)
    e = jnp.repeat(e, N, axis=2)
    h = jnp.concatenate((e, x1, x2, x3), axis=1)
    h = _block(h, p, "7")
    h = _block(h, p, "8")
    h = _block(h, p, "9")
    return _final_proj(h, p["W10"])


def kernel(x, params):
    return _forward(x, params)


# speedup vs baseline: 3.5393x; 1.0485x over previous
import functools

import jax
import jax.numpy as jnp
from jax import lax
from jax.experimental import pallas as pl
from jax.experimental.pallas import tpu as pltpu
from jax.experimental.pallas import tpu_sc as plsc

_K = 40
_NC, _NS = 2, 16
_NW = _NC * _NS
_BR = 256


def _sc_gather_call(table, idx, ch):
    M = idx.shape[0]
    R, Cp = table.shape
    m_per_w = M // _NW
    assert m_per_w % ch == 0 and m_per_w % 8 == 0
    n_steps = m_per_w // ch
    mesh = plsc.VectorSubcoreMesh(core_axis_name="c", subcore_axis_name="s",
                                  num_cores=_NC, num_subcores=_NS)

    @functools.partial(
        pl.kernel,
        out_type=jax.ShapeDtypeStruct((M, Cp), jnp.float32),
        mesh=mesh,
        scratch_types=[
            pltpu.VMEM((ch,), jnp.int32),
            pltpu.VMEM((ch, Cp), jnp.float32),
            pltpu.SemaphoreType.DMA,
        ],
        compiler_params=pltpu.CompilerParams(use_tc_tiling_on_sc=False),
    )
    def gather_k(table_hbm, idx_hbm, out_hbm, idx_v, rows_v, sem):
        wid = lax.axis_index("s") * _NC + lax.axis_index("c")
        w_base = wid * m_per_w

        def step(i, carry):
            base = w_base + i * ch
            pltpu.sync_copy(idx_hbm.at[pl.ds(base, ch)], idx_v)
            pltpu.async_copy(table_hbm.at[idx_v], rows_v, sem).wait()
            pltpu.sync_copy(rows_v, out_hbm.at[pl.ds(base, ch)])
            return carry

        lax.fori_loop(0, n_steps, step, 0)

    return gather_k(table, idx)


def _bisect_body(pd_ref, pmat_ref, o_ref):
    pd = pd_ref[...]
    kb = jax.lax.bitcast_convert_type(pd, jnp.int32)
    key = jnp.where(kb < 0, kb ^ jnp.int32(0x7FFFFFFF), kb)

    def it(_, lohi):
        lo, hi = lohi
        mid = (lo & hi) + ((lo ^ hi) >> 1)
        cnt = jnp.sum((key > mid).astype(jnp.int32), axis=1, keepdims=True)
        ge = cnt >= _K
        return jnp.where(ge, mid, lo), jnp.where(ge, hi, mid)

    lo0 = jnp.full((_BR, 1), jnp.iinfo(jnp.int32).min, jnp.int32)
    hi0 = jnp.full((_BR, 1), jnp.iinfo(jnp.int32).max, jnp.int32)
    _, hi = jax.lax.fori_loop(0, 32, it, (lo0, hi0))
    strict = (key > hi).astype(jnp.float32)
    eq = (key == hi).astype(jnp.float32)
    g = jnp.sum(strict, axis=1, keepdims=True).astype(jnp.int32)
    sw = jnp.dot(strict, pmat_ref[...],
                 preferred_element_type=jnp.float32).astype(jnp.int32)
    ew = jnp.dot(eq, pmat_ref[...],
                 preferred_element_type=jnp.float32).astype(jnp.int32)
    o_ref[...] = jnp.concatenate(
        [sw, ew, jnp.broadcast_to(g, (_BR, 128))], axis=1)


def _sc_emit_call(words):
    R = words.shape[0]
    rows_per_w = R // _NW
    RCH = 16
    n_ch = rows_per_w // RCH
    mesh = plsc.VectorSubcoreMesh(core_axis_name="c", subcore_axis_name="s",
                                  num_cores=_NC, num_subcores=_NS)

    @functools.partial(
        pl.kernel,
        out_type=jax.ShapeDtypeStruct((R, 48), jnp.int32),
        mesh=mesh,
        scratch_types=[
            pltpu.VMEM((RCH, 384), jnp.int32),
            pltpu.VMEM((RCH, 48), jnp.int32),
        ],
        compiler_params=pltpu.CompilerParams(use_tc_tiling_on_sc=False,
                                             needs_layout_passes=False),
    )
    def emit_k(words_hbm, out_hbm, win_v, out_v):
        wid = lax.axis_index("s") * _NC + lax.axis_index("c")
        w_base = wid * rows_per_w
        lane = lax.iota(jnp.int32, 16)

        def decode_pass(r, col_off, cnt0, limit):
            def grp(g, cnt):
                w0 = win_v[r, pl.ds(col_off + g * 16, 16)]
                colbase = (g * 16 + lane) * 16

                def popc(u):
                    v = u - ((u >> 1) & 0x5555)
                    v = (v & 0x3333) + ((v >> 2) & 0x3333)
                    v = (v + (v >> 4)) & 0x0F0F
                    return (v + (v >> 8)) & 0x1F

                def with_bits():
                    wcnt = popc(w0)
                    prefix = cnt + plsc.cumsum(wcnt) - wcnt
                    acc = jnp.zeros((16,), jnp.int32)
                    for b in range(16):
                        mc = (w0 >> b) & 1
                        pos = prefix + acc
                        plsc.store_scatter(
                            out_v,
                            [jnp.full((16,), r, jnp.int32), pos],
                            colbase + b,
                            mask=jnp.logical_and(mc != 0, pos < limit))
                        acc = acc + mc
                    return cnt + jnp.sum(wcnt)

                nz = jnp.sum((w0 != 0).astype(jnp.int32))
                return lax.cond(nz > 0, with_bits, lambda: cnt)

            return lax.fori_loop(0, 8, grp, cnt0)

        def chunk(ci, carry):
            base = w_base + ci * RCH
            pltpu.sync_copy(words_hbm.at[pl.ds(base, RCH)], win_v)

            def row(r, carry2):
                g_r = win_v[r, pl.ds(256, 16)][0]
                decode_pass(r, 0, 0, jnp.int32(48))
                decode_pass(r, 128, g_r, jnp.int32(_K))
                return carry2

            lax.fori_loop(0, RCH, row, 0)
            pltpu.sync_copy(out_v, out_hbm.at[pl.ds(base, RCH)])
            return carry

        lax.fori_loop(0, n_ch, chunk, 0)

    return emit_k(words)


def _knn(x, k):
    B, C, N = x.shape
    inner = -2.0 * jnp.matmul(jnp.swapaxes(x, 2, 1), x)
    xx = jnp.sum(x ** 2, axis=1, keepdims=True)
    pd = (-xx - inner - jnp.swapaxes(xx, 2, 1)).reshape(B * N, N)
    NB = (B * N) // _BR
    col = jax.lax.broadcasted_iota(jnp.int32, (N, 128), 0)
    wrd = jax.lax.broadcasted_iota(jnp.int32, (N, 128), 1)
    pmat = jnp.where(col // 16 == wrd, 1 << (col % 16), 0).astype(jnp.float32)
    words = pl.pallas_call(
        _bisect_body,
        grid=(NB,),
        in_specs=[
            pl.BlockSpec((_BR, N), lambda r: (r, 0)),
            pl.BlockSpec((N, 128), lambda r: (0, 0)),
        ],
        out_specs=pl.BlockSpec((_BR, 384), lambda r: (r, 0)),
        out_shape=jax.ShapeDtypeStruct((B * N, 384), jnp.int32),
    )(pd, pmat)
    idx = _sc_emit_call(words)[:, :k]
    v = jnp.take_along_axis(pd, idx, axis=1)
    kb2 = jax.lax.bitcast_convert_type(v, jnp.int32)
    vkey = jnp.where(kb2 < 0, kb2 ^ jnp.int32(0x7FFFFFFF), kb2)
    _, idx = jax.lax.sort((-vkey, idx), dimension=1, num_keys=2,
                          is_stable=False)
    return idx.reshape(B, N, k)


def _bn(x, g, b, eps=1e-5):
    axes = (0,) + tuple(range(2, x.ndim))
    m = jnp.mean(x, axis=axes, keepdims=True)
    v = jnp.var(x, axis=axes, keepdims=True)
    shape = (1, -1) + (1,) * (x.ndim - 2)
    return g.reshape(shape) * (x - m) / jnp.sqrt(v + eps) + b.reshape(shape)


def _conv(x, W):
    if x.ndim == 3:
        return jnp.einsum("oi,bin->bon", W, x)
    return jnp.einsum("oi,bihw->bohw", W, x)


def _block(x, p, name):
    y = _conv(x, p["W" + name])
    y = _bn(y, p["g" + name], p["b" + name])
    return jax.nn.leaky_relu(y, 0.2)


def _graph_feature(x, k):
    B, C, N = x.shape
    idx = _knn(x, k)
    xt = jnp.swapaxes(x, 2, 1)
    idx_base = jnp.arange(B).reshape(-1, 1, 1) * N
    flat_idx = (idx + idx_base).reshape(-1).astype(jnp.int32)
    Cp = max(16, ((C + 15) // 16) * 16)
    table = jnp.zeros((B * N, Cp), jnp.float32).at[:, :C].set(
        xt.reshape(B * N, C))
    feature = _sc_gather_call(table, flat_idx, 1024)[:, :C]
    feature = feature.reshape(B, N, k, C)
    xe = jnp.broadcast_to(xt[:, :, None, :], (B, N, k, C))
    f = jnp.concatenate((feature - xe, xe), axis=3)
    return jnp.transpose(f, (0, 3, 1, 2))


def _final_proj_kernel(h_ref, w_ref, o_ref):
    o_ref[0] = jnp.dot(w_ref[...], h_ref[0],
                       preferred_element_type=jnp.float32)


def _final_proj(h, W):
    B, C, N = h.shape
    O = W.shape[0]
    Opad = 56
    Wp = jnp.zeros((Opad, C), W.dtype).at[:O].set(W)
    out = pl.pallas_call(
        _final_proj_kernel,
        grid=(B,),
        in_specs=[
            pl.BlockSpec((1, C, N), lambda b: (b, 0, 0)),
            pl.BlockSpec((Opad, C), lambda b: (0, 0)),
        ],
        out_specs=pl.BlockSpec((1, Opad, N), lambda b: (b, 0, 0)),
        out_shape=jax.ShapeDtypeStruct((B, Opad, N), jnp.float32),
    )(h, Wp)
    return out[:, :O, :]


def _forward(x, p):
    B, _, N = x.shape
    x2z = _block(x[:, :2, :], p, "0_0") * x[:, 2, :][:, None, :]
    x2y = _block(x[:, jnp.array([0, 2]), :], p, "0_1") * x[:, 1, :][:, None, :]
    x2x = _block(x[:, 1:3, :], p, "0_1") * x[:, 0, :][:, None, :]
    h = jnp.concatenate((x, x2x, x2y, x2z), axis=1)
    f = _graph_feature(h, _K)
    f = _block(f, p, "1")
    f = _block(f, p, "2")
    x1 = jnp.max(f, axis=-1)
    f = _graph_feature(x1, _K)
    f = _block(f, p, "3")
    f = _block(f, p, "4")
    x2 = jnp.max(f, axis=-1)
    f = _graph_feature(x2, _K)
    f = _block(f, p, "5")
    x3 = jnp.max(f, axis=-1)
    c = jnp.concatenate((x1, x2, x3), axis=1)
    e = _block(c, p, "6")
    e = jnp.max(e, axis=-1, keepdims=True)
    e = jnp.repeat(e, N, axis=2)
    h = jnp.concatenate((e, x1, x2, x3), axis=1)
    h = _block(h, p, "7")
    h = _block(h, p, "8")
    h = _block(h, p, "9")
    return _final_proj(h, p["W10"])


def kernel(x, params):
    return _forward(x, params)
